# baseline (speedup 1.0000x reference)
import jax
import jax.numpy as jnp
from jax import lax
from jax.experimental import pallas as pl
from jax.experimental.pallas import tpu as pltpu
from jax.experimental.pallas import tpu_sc as plsc

NUM_CORES = 2
NUM_SUBCORES = 16
LANES = 16
NUM_WORKERS = NUM_CORES * NUM_SUBCORES

BATCH = 16384
DIM = 32
PACK = 4
SHIFT = PACK.bit_length() - 1
MINOR = DIM * PACK
BPW = BATCH // NUM_WORKERS
QUARTER = 128
NQ = BPW // QUARTER
GROUPS = QUARTER // LANES


def _score_body(h_hbm, r_hbm, t_hbm, ent_hbm, rel_hbm, out_hbm,
                h_v, r_v, t_v, h4_v, r4_v, t4_v,
                he0, re0, te0, he1, re1, te1, out_v,
                sem0, sem1):
    wid = lax.axis_index("s") * NUM_CORES + lax.axis_index("c")
    base = wid * BPW
    pltpu.sync_copy(h_hbm.at[pl.ds(base, BPW)], h_v)
    pltpu.sync_copy(r_hbm.at[pl.ds(base, BPW)], r_v)
    pltpu.sync_copy(t_hbm.at[pl.ds(base, BPW)], t_v)

    for k in range(BPW // LANES):
        s = pl.ds(k * LANES, LANES)
        h4_v[s] = h_v[s] >> SHIFT
        r4_v[s] = r_v[s] >> SHIFT
        t4_v[s] = t_v[s] >> SHIFT

    bufs = [(he0, re0, te0), (he1, re1, te1)]
    sems = [sem0, sem1]

    def fire(q):
        sl = pl.ds(q * QUARTER, QUARTER)
        he, re, te = bufs[q % 2]
        sem = sems[q % 2]
        return [
            pltpu.async_copy(ent_hbm.at[h4_v.at[sl]], he, sem),
            pltpu.async_copy(rel_hbm.at[r4_v.at[sl]], re, sem),
            pltpu.async_copy(ent_hbm.at[t4_v.at[sl]], te, sem),
        ]

    iota = lax.iota(jnp.int32, LANES)
    inflight = {0: fire(0)}

    for q in range(NQ):
        if q + 1 < NQ:
            inflight[q + 1] = fire(q + 1)
        for cp in inflight.pop(q):
            cp.wait()
        he, re, te = bufs[q % 2]

        def group(g, carry, q=q, he=he, re=re, te=te):
            off = pl.multiple_of(q * QUARTER + g * LANES, LANES)
            sl = pl.ds(off, LANES)
            h16 = h_v[sl]
            r16 = r_v[sl]
            t16 = t_v[sl]
            offh = (h16 & (PACK - 1)) << 5
            offr = (r16 & (PACK - 1)) << 5
            offt = (t16 & (PACK - 1)) << 5
            row = iota + g * LANES
            acc = jnp.zeros((LANES,), jnp.float32)
            for j in range(DIM):
                colj = (iota + j) & (DIM - 1)
                hv = plsc.load_gather(he, [row, offh + colj])
                rv = plsc.load_gather(re, [row, offr + colj])
                tv = plsc.load_gather(te, [row, offt + colj])
                d = hv + rv - tv
                acc = acc + d * d
            x = acc + 1e-12
            i = plsc.bitcast(x, jnp.int32)
            i = jnp.int32(0x5F3759DF) - (i >> 1)
            y = plsc.bitcast(i, jnp.float32)
            for _ in range(3):
                y = y * (1.5 - 0.5 * x * y * y)
            out_v[sl] = -(x * y)
            return carry

        lax.fori_loop(0, GROUPS, group, 0)

    pltpu.sync_copy(out_v, out_hbm.at[pl.ds(base, BPW)])


def kernel(h, r, t, ent_emb, rel_emb):
    h = h.astype(jnp.int32)
    r = r.astype(jnp.int32)
    t = t.astype(jnp.int32)
    ent4 = ent_emb.reshape(ent_emb.shape[0] // PACK, MINOR)
    rel4 = rel_emb.reshape(rel_emb.shape[0] // PACK, MINOR)
    mesh = plsc.VectorSubcoreMesh(core_axis_name="c", subcore_axis_name="s")
    fn = pl.kernel(
        _score_body,
        mesh=mesh,
        compiler_params=pltpu.CompilerParams(
            needs_layout_passes=False, use_tc_tiling_on_sc=True
        ),
        out_type=jax.ShapeDtypeStruct((BATCH,), jnp.float32),
        scratch_types=[
            pltpu.VMEM((BPW,), jnp.int32),
            pltpu.VMEM((BPW,), jnp.int32),
            pltpu.VMEM((BPW,), jnp.int32),
            pltpu.VMEM((BPW,), jnp.int32),
            pltpu.VMEM((BPW,), jnp.int32),
            pltpu.VMEM((BPW,), jnp.int32),
            pltpu.VMEM((QUARTER, MINOR), jnp.float32),
            pltpu.VMEM((QUARTER, MINOR), jnp.float32),
            pltpu.VMEM((QUARTER, MINOR), jnp.float32),
            pltpu.VMEM((QUARTER, MINOR), jnp.float32),
            pltpu.VMEM((QUARTER, MINOR), jnp.float32),
            pltpu.VMEM((QUARTER, MINOR), jnp.float32),
            pltpu.VMEM((BPW,), jnp.float32),
            pltpu.SemaphoreType.DMA,
            pltpu.SemaphoreType.DMA,
        ],
    )
    return fn(h, r, t, ent4, rel4)
